# Initial kernel scaffold; baseline (speedup 1.0000x reference)
import jax
import jax.numpy as jnp
from jax.experimental import pallas as pl


def kernel(seq1, neg, tmp, edge_index, msk, samp_bias1, samp_bias2, subgraph, params):
    raise NotImplementedError("write your pallas kernel here")


# speedup vs baseline: 1.0064x; 1.0064x over previous
import jax
import jax.numpy as jnp
from jax.experimental import pallas as pl
from jax.experimental.pallas import tpu as pltpu

N = 10000
D = 128
H = 128
W = 10


def _lstm_steps(x_seq, Wih, Whh, b, keep_seq):
    n = x_seq.shape[0]
    h = jnp.zeros((n, H), jnp.float32)
    c = jnp.zeros((n, H), jnp.float32)
    hs = []
    for t in range(W):
        z = x_seq[:, t, :] @ Wih + h @ Whh + b
        i, f, g, o = jnp.split(z, 4, axis=-1)
        c = jax.nn.sigmoid(f) * c + jax.nn.sigmoid(i) * jnp.tanh(g)
        h = jax.nn.sigmoid(o) * jnp.tanh(c)
        if keep_seq:
            hs.append(h)
    return h, (jnp.stack(hs, axis=1) if keep_seq else None)


def _mlp3(x, W1, b1, W2, b2, W3, b3):
    h = jax.nn.relu(x @ W1 + b1)
    h = jax.nn.relu(h @ W2 + b2)
    return h @ W3 + b3


def _scores_body(hv_ref, bias_ref, out_ref):
    out_ref[...] = hv_ref[...] + bias_ref[...]


def kernel(seq1, neg, tmp, edge_index, msk, samp_bias1, samp_bias2, subgraph, params):
    p = params
    src, dst = edge_index[0], edge_index[1]

    x_sub = seq1[subgraph]
    h1, hs1 = _lstm_steps(x_sub, p["Wih1"], p["Whh1"], p["b1"], keep_seq=True)
    h2, _ = _lstm_steps(hs1, p["Wih2"], p["Whh2"], p["b2"], keep_seq=False)
    c_out = jax.nn.sigmoid(jnp.mean(h2, axis=0))

    x_neg = seq1[neg]
    h_neg, _ = _lstm_steps(x_neg, p["Wih1"], p["Whh1"], p["b1"], keep_seq=False)

    deg = jax.ops.segment_sum(jnp.ones((src.shape[0],), jnp.float32), dst, num_segments=N)
    degc = jnp.clip(deg, 1.0, None)[:, None]
    t1 = seq1 @ p["Wg1"]
    f1 = jax.nn.relu(jax.ops.segment_sum(t1[src], dst, num_segments=N) / degc)
    t2 = f1 @ p["Wg2"]
    f2 = jax.nn.relu(jax.ops.segment_sum(t2[src], dst, num_segments=N) / degc)

    g = f2[tmp].reshape(N, W * D)
    pat = jax.nn.relu(g @ p["Wl1"] + p["bl1"])
    pat = jax.nn.relu(pat @ p["Wl2"] + p["bl2"])

    new_input = jnp.concatenate([h1, seq1, pat], axis=1)
    fea = _mlp3(new_input, p["Wa1"], p["ba1"], p["Wa2"], p["ba2"], p["Wa3"], p["ba3"])

    v = p["Wd"] @ c_out
    hv = jnp.stack([h1 @ v, h_neg @ v], axis=0)
    bias = jnp.stack([samp_bias1, samp_bias2], axis=0)
    scores = pl.pallas_call(
        _scores_body,
        out_shape=jax.ShapeDtypeStruct((2, N), jnp.float32),
    )(hv, bias)
    ret = scores.reshape(2 * N)

    neighbor_sim = jnp.einsum("nd,nwd->nw", seq1, x_sub)
    nb_dec = _mlp3(h1, p["Wls1"], p["bls1"], p["Wls2"], p["bls2"], p["Wls3"], p["bls3"])
    feature_loss3 = jnp.mean((neighbor_sim - nb_dec) ** 2)
    feature_loss = jnp.mean((seq1 - _mlp3(h1, p["Wf1"], p["bf1"], p["Wf2"], p["bf2"], p["Wf3"], p["bf3"])) ** 2)
    feature_loss2 = jnp.mean((seq1 - _mlp3(fea, p["W2f1"], p["b2f1"], p["W2f2"], p["b2f2"], p["W2f3"], p["b2f3"])) ** 2)
    total = feature_loss + feature_loss2 + 1e-07 * feature_loss3
    return ret, total


# speedup vs baseline: 1.9048x; 1.8927x over previous
import functools

import jax
import jax.numpy as jnp
from jax import lax
from jax.experimental import pallas as pl
from jax.experimental.pallas import tpu as pltpu
from jax.experimental.pallas import tpu_sc as plsc

N = 10000
D = 128
H = 128
W = 10
E = 320000

_NC = 2
_NS = 16
_LANES = 16
_DH = D // _NC
_EW = E // _NS
_ECHUNK = 400
_ENCHUNK = _EW // _ECHUNK
_NPAD = 10240
_RPT = _NPAD // _NS


def _segsum_body(h_hbm, src_hbm, dst_hbm, zrow_hbm, zdeg_hbm, ones_hbm,
                 agg_hbm, deg_hbm,
                 src_v, dst_v, rows_v, ones_v, acc_sh, hist_sh, sem):
    c = lax.axis_index("c")
    s = lax.axis_index("s")
    pltpu.sync_copy(zrow_hbm, acc_sh.at[pl.ds(s * _RPT, _RPT)])
    pltpu.sync_copy(zdeg_hbm, hist_sh.at[pl.ds(s * _RPT, _RPT)])
    pltpu.sync_copy(ones_hbm, ones_v)
    plsc.subcore_barrier()

    def chunk(i, carry):
        base = s * _EW + i * _ECHUNK
        pltpu.sync_copy(src_hbm.at[pl.ds(base, _ECHUNK)], src_v)
        pltpu.sync_copy(dst_hbm.at[pl.ds(base, _ECHUNK)], dst_v)
        pltpu.async_copy(h_hbm.at[c].at[src_v], rows_v, sem).wait()
        pltpu.sync_copy(rows_v, acc_sh.at[dst_v], add=True)

        @pl.when(c == 0)
        def _():
            pltpu.sync_copy(ones_v, hist_sh.at[dst_v], add=True)

        return carry

    lax.fori_loop(0, _ENCHUNK, chunk, 0)
    plsc.subcore_barrier()
    pltpu.sync_copy(acc_sh.at[pl.ds(s * _RPT, _RPT)],
                    agg_hbm.at[c, pl.ds(s * _RPT, _RPT)])

    @pl.when(c == 0)
    def _():
        pltpu.sync_copy(hist_sh.at[pl.ds(s * _RPT, _RPT)],
                        deg_hbm.at[pl.ds(s * _RPT, _RPT)])


@jax.jit
def _sc_segment_sum(h, src, dst):
    mesh = plsc.VectorSubcoreMesh(core_axis_name="c", subcore_axis_name="s")
    kern = pl.kernel(
        _segsum_body,
        mesh=mesh,
        compiler_params=pltpu.CompilerParams(use_tc_tiling_on_sc=False),
        out_type=(
            jax.ShapeDtypeStruct((_NC, _NPAD, _DH), jnp.float32),
            jax.ShapeDtypeStruct((_NPAD, _LANES), jnp.float32),
        ),
        scratch_types=[
            pltpu.VMEM((_ECHUNK,), jnp.int32),
            pltpu.VMEM((_ECHUNK,), jnp.int32),
            pltpu.VMEM((_ECHUNK, _DH), jnp.float32),
            pltpu.VMEM((_ECHUNK, _LANES), jnp.float32),
            pltpu.VMEM_SHARED((_NPAD, _DH), jnp.float32),
            pltpu.VMEM_SHARED((_NPAD, _LANES), jnp.float32),
            pltpu.SemaphoreType.DMA,
        ],
    )
    hsplit = jnp.stack([h[:, :_DH], h[:, _DH:]])
    zrow = jnp.zeros((_RPT, _DH), jnp.float32)
    zdeg = jnp.zeros((_RPT, _LANES), jnp.float32)
    ones = jnp.ones((_ECHUNK, _LANES), jnp.float32)
    return kern(hsplit, src, dst, zrow, zdeg, ones)


def _lstm_steps(x_seq, Wih, Whh, b, keep_seq):
    n = x_seq.shape[0]
    h = jnp.zeros((n, H), jnp.float32)
    c = jnp.zeros((n, H), jnp.float32)
    hs = []
    for t in range(W):
        z = x_seq[:, t, :] @ Wih + h @ Whh + b
        i, f, g, o = jnp.split(z, 4, axis=-1)
        c = jax.nn.sigmoid(f) * c + jax.nn.sigmoid(i) * jnp.tanh(g)
        h = jax.nn.sigmoid(o) * jnp.tanh(c)
        if keep_seq:
            hs.append(h)
    return h, (jnp.stack(hs, axis=1) if keep_seq else None)


def _mlp3(x, W1, b1, W2, b2, W3, b3):
    h = jax.nn.relu(x @ W1 + b1)
    h = jax.nn.relu(h @ W2 + b2)
    return h @ W3 + b3


def _scores_body(hv_ref, bias_ref, out_ref):
    out_ref[...] = hv_ref[...] + bias_ref[...]


def kernel(seq1, neg, tmp, edge_index, msk, samp_bias1, samp_bias2, subgraph, params):
    p = params
    src, dst = edge_index[0], edge_index[1]

    x_sub = seq1[subgraph]
    h1, hs1 = _lstm_steps(x_sub, p["Wih1"], p["Whh1"], p["b1"], keep_seq=True)
    h2, _ = _lstm_steps(hs1, p["Wih2"], p["Whh2"], p["b2"], keep_seq=False)
    c_out = jax.nn.sigmoid(jnp.mean(h2, axis=0))

    x_neg = seq1[neg]
    h_neg, _ = _lstm_steps(x_neg, p["Wih1"], p["Whh1"], p["b1"], keep_seq=False)

    t1 = seq1 @ p["Wg1"]
    agg1p, degp = _sc_segment_sum(t1, src, dst)
    degc = jnp.clip(degp[:N, 0], 1.0, None)[:, None]
    f1 = jax.nn.relu(jnp.concatenate([agg1p[0, :N], agg1p[1, :N]], axis=1) / degc)
    t2 = f1 @ p["Wg2"]
    agg2p, _ = _sc_segment_sum(t2, src, dst)
    f2 = jax.nn.relu(jnp.concatenate([agg2p[0, :N], agg2p[1, :N]], axis=1) / degc)

    g = f2[tmp].reshape(N, W * D)
    pat = jax.nn.relu(g @ p["Wl1"] + p["bl1"])
    pat = jax.nn.relu(pat @ p["Wl2"] + p["bl2"])

    new_input = jnp.concatenate([h1, seq1, pat], axis=1)
    fea = _mlp3(new_input, p["Wa1"], p["ba1"], p["Wa2"], p["ba2"], p["Wa3"], p["ba3"])

    v = p["Wd"] @ c_out
    hv = jnp.stack([h1 @ v, h_neg @ v], axis=0)
    bias = jnp.stack([samp_bias1, samp_bias2], axis=0)
    scores = pl.pallas_call(
        _scores_body,
        out_shape=jax.ShapeDtypeStruct((2, N), jnp.float32),
    )(hv, bias)
    ret = scores.reshape(2 * N)

    neighbor_sim = jnp.einsum("nd,nwd->nw", seq1, x_sub)
    nb_dec = _mlp3(h1, p["Wls1"], p["bls1"], p["Wls2"], p["bls2"], p["Wls3"], p["bls3"])
    feature_loss3 = jnp.mean((neighbor_sim - nb_dec) ** 2)
    feature_loss = jnp.mean((seq1 - _mlp3(h1, p["Wf1"], p["bf1"], p["Wf2"], p["bf2"], p["Wf3"], p["bf3"])) ** 2)
    feature_loss2 = jnp.mean((seq1 - _mlp3(fea, p["W2f1"], p["b2f1"], p["W2f2"], p["b2f2"], p["W2f3"], p["b2f3"])) ** 2)
    total = feature_loss + feature_loss2 + 1e-07 * feature_loss3
    return ret, total
